# baseline (speedup 1.0000x reference)
import functools

import jax
import jax.numpy as jnp
from jax import lax
from jax.experimental import pallas as pl
from jax.experimental.pallas import tpu as pltpu


def _round_up(x, m):
    return ((x + m - 1) // m) * m


def _pick_chunk(T, max_chunk=8):
    if T <= max_chunk:
        return T
    for c in range(max_chunk, 0, -1):
        if T % c == 0:
            return c
    return T


def _lstm_cls_kernel(t_len, ids_ref, emb_ref, wih_ref, whh_ref, b_ref,
                     wcls_ref, bcls_ref, out_ref, h_ref, c_ref,
                     xlo_ref, xhi_ref):
    b = pl.program_id(0)
    t = pl.program_id(1)
    n_t = pl.num_programs(1)
    TB, Dp = h_ref.shape
    CHUNK = xlo_ref.shape[1] // TB
    H = TB // 2
    DL = Dp // 2

    @pl.when(t == 0)
    def _init():
        h_ref[...] = jnp.zeros_like(h_ref)
        c_ref[...] = jnp.zeros_like(c_ref)

    def gather_step(s, buf, fb0_):
        fb = fb0_ + s
        sTB = pl.multiple_of(s * TB, 8)
        for u in range(8):
            for g in range(TB // 8):
                j = g * 8 + u
                idx = ids_ref[fb + j * t_len]
                row0 = pl.multiple_of((idx >> 3) << 3, 8)
                ch = emb_ref[pl.ds(row0, 8), :]
                sh = u - (idx & 7)
                lo = pltpu.roll(ch[:, :DL], sh, axis=0)
                hi = pltpu.roll(ch[:, DL:], sh, axis=0)
                xlo_ref[buf, pl.ds(sTB + j, 1), :] = lo[u:u + 1, :]
                xhi_ref[buf, pl.ds(sTB + j, 1), :] = hi[u:u + 1, :]

    fb0_self = b * TB * t_len + t * CHUNK

    @pl.when(t == 0)
    def _prologue():
        lax.fori_loop(0, CHUNK,
                      lambda s, z: (gather_step(s, 0, fb0_self), z)[1], 0)

    wih = wih_ref[...]
    whh = whh_ref[...]
    bias = b_ref[...]
    cur = t & 1
    nxt = (t + 1) & 1
    tn = jnp.minimum(t + 1, n_t - 1)
    fb0_next = b * TB * t_len + tn * CHUNK

    def half_step(x_t, h, c):
        gates = (jnp.dot(x_t, wih, preferred_element_type=jnp.float32)
                 + jnp.dot(h, whh, preferred_element_type=jnp.float32)
                 + bias)
        sig = jax.nn.sigmoid(gates[:, :3 * Dp])
        g_g = jnp.tanh(gates[:, 3 * Dp:])
        i_g = sig[:, :Dp]
        f_g = sig[:, Dp:2 * Dp]
        o_g = sig[:, 2 * Dp:]
        c_new = f_g * c + i_g * g_g
        h_new = (o_g * jnp.tanh(c_new)).astype(jnp.bfloat16)
        return h_new, c_new

    def step(s, carry):
        hA, cA, hB, cB = carry
        base = pl.multiple_of(s * TB, 8)
        xA = jnp.concatenate([xlo_ref[cur, pl.ds(base, H), :],
                              xhi_ref[cur, pl.ds(base, H), :]], axis=1)
        xB = jnp.concatenate([xlo_ref[cur, pl.ds(base + H, H), :],
                              xhi_ref[cur, pl.ds(base + H, H), :]], axis=1)
        hA, cA = half_step(xA, hA, cA)
        hB, cB = half_step(xB, hB, cB)
        gather_step(s, nxt, fb0_next)
        return hA, cA, hB, cB

    carry0 = (h_ref[pl.ds(0, H), :], c_ref[pl.ds(0, H), :],
              h_ref[pl.ds(H, H), :], c_ref[pl.ds(H, H), :])
    hA, cA, hB, cB = jax.lax.fori_loop(0, CHUNK, step, carry0)
    h_ref[pl.ds(0, H), :] = hA
    c_ref[pl.ds(0, H), :] = cA
    h_ref[pl.ds(H, H), :] = hB
    c_ref[pl.ds(H, H), :] = cB

    @pl.when(t == n_t - 1)
    def _finish():
        wcls = wcls_ref[...]
        bcls = bcls_ref[...]
        out_ref[pl.ds(0, H), :] = (
            jnp.dot(hA, wcls, preferred_element_type=jnp.float32) + bcls
        ).astype(out_ref.dtype)
        out_ref[pl.ds(H, H), :] = (
            jnp.dot(hB, wcls, preferred_element_type=jnp.float32) + bcls
        ).astype(out_ref.dtype)


def kernel(x_ids, embedding, w_ih_T, w_hh_T, b_lstm, w_cls_T, b_cls):
    V, Dp = embedding.shape
    G = w_ih_T.shape[1]
    Cp = w_cls_T.shape[1]
    B, T = x_ids.shape

    if B % 256 == 0:
        TB = 256
    elif B % 16 == 0:
        TB = B
    else:
        TB = _round_up(B, 16)
    Bp = _round_up(B, TB)
    nb = Bp // TB
    CHUNK = _pick_chunk(T)
    nt = T // CHUNK

    emb_f32 = embedding.astype(jnp.float32)

    ids = x_ids
    if Bp != B:
        ids = jnp.pad(ids, ((0, Bp - B), (0, 0)))
    ids = ids.reshape(-1)

    full = lambda shape: pl.BlockSpec(shape, lambda b, t: tuple(0 for _ in shape))

    out = pl.pallas_call(
        functools.partial(_lstm_cls_kernel, T),
        out_shape=jax.ShapeDtypeStruct((Bp, Cp), jnp.float32),
        grid=(nb, nt),
        in_specs=[
            pl.BlockSpec(memory_space=pltpu.SMEM),
            full((V, Dp)),
            full((Dp, G)),
            full((Dp, G)),
            full((1, G)),
            full((Dp, Cp)),
            full((1, Cp)),
        ],
        out_specs=pl.BlockSpec((TB, Cp), lambda b, t: (b, 0)),
        scratch_shapes=[
            pltpu.VMEM((TB, Dp), jnp.bfloat16),
            pltpu.VMEM((TB, Dp), jnp.float32),
            pltpu.VMEM((2, CHUNK * TB, Dp // 2), jnp.float32),
            pltpu.VMEM((2, CHUNK * TB, Dp // 2), jnp.float32),
        ],
        compiler_params=pltpu.CompilerParams(
            dimension_semantics=("parallel", "arbitrary"),
            vmem_limit_bytes=64 * 1024 * 1024,
        ),
    )(ids, emb_f32, w_ih_T, w_hh_T, b_lstm, w_cls_T, b_cls)
    return out[:B, :128]


# speedup vs baseline: 1.2926x; 1.2926x over previous
import jax
import jax.numpy as jnp
from jax import lax
from jax.experimental import pallas as pl
from jax.experimental.pallas import tpu as pltpu


def _round_up(x, m):
    return ((x + m - 1) // m) * m


def _pick_chunk(T, max_chunk=16):
    if T <= max_chunk:
        return T
    for c in range(max_chunk, 0, -1):
        if T % c == 0:
            return c
    return T


def _lstm_cls_kernel(t_len, ids_ref, emb_ref, wih_ref, whh_ref, b_ref,
                     wcls_ref, bcls_ref, out_ref, h_ref, c_ref,
                     xlo_ref, xhi_ref):
    b = pl.program_id(0)
    t = pl.program_id(1)
    n_t = pl.num_programs(1)
    TB, Dp = h_ref.shape
    CHUNK = xlo_ref.shape[0] // TB
    H = TB // 2
    DL = Dp // 2

    @pl.when(t == 0)
    def _init():
        h_ref[...] = jnp.zeros_like(h_ref)
        c_ref[...] = jnp.zeros_like(c_ref)

    n_b = pl.num_programs(0)
    fb0 = (t * CHUNK) * (n_b * TB) + b * TB

    def g_body(s, _):
        fb = fb0 + s * (n_b * TB)
        sTB = pl.multiple_of(s * TB, 8)
        for u in range(8):
          for g in range(TB // 8):
            j = g * 8 + u
            idx = ids_ref[fb + j]
            row0 = pl.multiple_of((idx >> 3) << 3, 8)
            ch = emb_ref[pl.ds(row0, 8), :]
            sh = u - (idx & 7)
            lo = pltpu.roll(ch[:, :DL], sh, axis=0)
            hi = pltpu.roll(ch[:, DL:], sh, axis=0)
            xlo_ref[pl.ds(sTB + j, 1), :] = lo[u:u + 1, :]
            xhi_ref[pl.ds(sTB + j, 1), :] = hi[u:u + 1, :]
        return 0

    lax.fori_loop(0, CHUNK, g_body, 0)

    wih = wih_ref[...]
    whh = whh_ref[...]
    bias = b_ref[...]

    def half_step(x_t, h, c):
        gates = (jnp.dot(x_t, wih, preferred_element_type=jnp.float32)
                 + jnp.dot(h, whh, preferred_element_type=jnp.float32)
                 + bias)
        sig = jax.nn.sigmoid(gates[:, :3 * Dp])
        g_g = jnp.tanh(gates[:, 3 * Dp:])
        i_g = sig[:, :Dp]
        f_g = sig[:, Dp:2 * Dp]
        o_g = sig[:, 2 * Dp:]
        c_new = f_g * c + i_g * g_g
        h_new = (o_g * jnp.tanh(c_new)).astype(jnp.bfloat16)
        return h_new, c_new

    def step(s, carry):
        hA, cA, hB, cB = carry
        base = pl.multiple_of(s * TB, 8)
        xA = jnp.concatenate([xlo_ref[pl.ds(base, H), :],
                              xhi_ref[pl.ds(base, H), :]], axis=1)
        xB = jnp.concatenate([xlo_ref[pl.ds(base + H, H), :],
                              xhi_ref[pl.ds(base + H, H), :]], axis=1)
        hA, cA = half_step(xA, hA, cA)
        hB, cB = half_step(xB, hB, cB)
        return hA, cA, hB, cB

    carry0 = (h_ref[pl.ds(0, H), :], c_ref[pl.ds(0, H), :],
              h_ref[pl.ds(H, H), :], c_ref[pl.ds(H, H), :])
    hA, cA, hB, cB = jax.lax.fori_loop(0, CHUNK, step, carry0, unroll=2)
    h_ref[pl.ds(0, H), :] = hA
    c_ref[pl.ds(0, H), :] = cA
    h_ref[pl.ds(H, H), :] = hB
    c_ref[pl.ds(H, H), :] = cB

    @pl.when(t == n_t - 1)
    def _finish():
        wcls = wcls_ref[...]
        bcls = bcls_ref[...]
        out_ref[pl.ds(0, H), :] = (
            jnp.dot(hA, wcls, preferred_element_type=jnp.float32) + bcls
        ).astype(out_ref.dtype)
        out_ref[pl.ds(H, H), :] = (
            jnp.dot(hB, wcls, preferred_element_type=jnp.float32) + bcls
        ).astype(out_ref.dtype)


def kernel(x_ids, embedding, w_ih_T, w_hh_T, b_lstm, w_cls_T, b_cls):
    V, Dp = embedding.shape
    G = w_ih_T.shape[1]
    Cp = w_cls_T.shape[1]
    B, T = x_ids.shape

    if B % 256 == 0:
        TB = 256
    elif B % 16 == 0:
        TB = B
    else:
        TB = _round_up(B, 16)
    Bp = _round_up(B, TB)
    nb = Bp // TB
    CHUNK = _pick_chunk(T)
    nt = T // CHUNK

    emb_f32 = embedding.astype(jnp.float32)

    ids = x_ids
    if Bp != B:
        ids = jnp.pad(ids, ((0, Bp - B), (0, 0)))
    ids = ids.T.reshape(-1)

    full = lambda shape: pl.BlockSpec(shape, lambda b, t: tuple(0 for _ in shape))

    import functools
    out = pl.pallas_call(
        functools.partial(_lstm_cls_kernel, T),
        out_shape=jax.ShapeDtypeStruct((Bp, Cp), jnp.float32),
        grid=(nb, nt),
        in_specs=[
            pl.BlockSpec(memory_space=pltpu.SMEM),
            full((V, Dp)),
            full((Dp, G)),
            full((Dp, G)),
            full((1, G)),
            full((Dp, Cp)),
            full((1, Cp)),
        ],
        out_specs=pl.BlockSpec((TB, Cp), lambda b, t: (b, 0)),
        scratch_shapes=[
            pltpu.VMEM((TB, Dp), jnp.bfloat16),
            pltpu.VMEM((TB, Dp), jnp.float32),
            pltpu.VMEM((CHUNK * TB, Dp // 2), jnp.float32),
            pltpu.VMEM((CHUNK * TB, Dp // 2), jnp.float32),
        ],
        compiler_params=pltpu.CompilerParams(
            dimension_semantics=("parallel", "arbitrary"),
            vmem_limit_bytes=64 * 1024 * 1024,
        ),
    )(ids, emb_f32, w_ih_T, w_hh_T, b_lstm, w_cls_T, b_cls)
    return out[:B, :128]


# speedup vs baseline: 1.2955x; 1.0022x over previous
import jax
import jax.numpy as jnp
from jax import lax
from jax.experimental import pallas as pl
from jax.experimental.pallas import tpu as pltpu


def _round_up(x, m):
    return ((x + m - 1) // m) * m


def _pick_chunk(T, max_chunk=32):
    if T <= max_chunk:
        return T
    for c in range(max_chunk, 0, -1):
        if T % c == 0:
            return c
    return T


def _lstm_cls_kernel(t_len, ids_ref, emb_ref, wih_ref, whh_ref, b_ref,
                     wcls_ref, bcls_ref, out_ref, h_ref, c_ref,
                     xlo_ref, xhi_ref):
    b = pl.program_id(0)
    t = pl.program_id(1)
    n_t = pl.num_programs(1)
    TB, Dp = h_ref.shape
    CHUNK = xlo_ref.shape[0] // TB
    H = TB // 2
    DL = Dp // 2

    @pl.when(t == 0)
    def _init():
        h_ref[...] = jnp.zeros_like(h_ref)
        c_ref[...] = jnp.zeros_like(c_ref)

    n_b = pl.num_programs(0)
    fb0 = (t * CHUNK) * (n_b * TB) + b * TB

    def g_body(s, _):
        fb = fb0 + s * (n_b * TB)
        sTB = pl.multiple_of(s * TB, 8)
        for u in range(8):
          for g in range(TB // 8):
            j = g * 8 + u
            idx = ids_ref[fb + j]
            row0 = pl.multiple_of((idx >> 3) << 3, 8)
            ch = emb_ref[pl.ds(row0, 8), :]
            sh = u - (idx & 7)
            lo = pltpu.roll(ch[:, :DL], sh, axis=0)
            hi = pltpu.roll(ch[:, DL:], sh, axis=0)
            xlo_ref[pl.ds(sTB + j, 1), :] = lo[u:u + 1, :]
            xhi_ref[pl.ds(sTB + j, 1), :] = hi[u:u + 1, :]
        return 0

    lax.fori_loop(0, CHUNK, g_body, 0)

    wih = wih_ref[...]
    whh = whh_ref[...]
    bias = b_ref[...]

    def half_step(x_t, h, c):
        gates = (jnp.dot(x_t, wih, preferred_element_type=jnp.float32)
                 + jnp.dot(h, whh, preferred_element_type=jnp.float32)
                 + bias)
        sig = jax.nn.sigmoid(gates[:, :3 * Dp])
        g_g = jnp.tanh(gates[:, 3 * Dp:])
        i_g = sig[:, :Dp]
        f_g = sig[:, Dp:2 * Dp]
        o_g = sig[:, 2 * Dp:]
        c_new = f_g * c + i_g * g_g
        h_new = (o_g * jnp.tanh(c_new)).astype(jnp.bfloat16)
        return h_new, c_new

    def step(s, carry):
        hA, cA, hB, cB = carry
        base = pl.multiple_of(s * TB, 8)
        xA = jnp.concatenate([xlo_ref[pl.ds(base, H), :],
                              xhi_ref[pl.ds(base, H), :]], axis=1)
        xB = jnp.concatenate([xlo_ref[pl.ds(base + H, H), :],
                              xhi_ref[pl.ds(base + H, H), :]], axis=1)
        hA, cA = half_step(xA, hA, cA)
        hB, cB = half_step(xB, hB, cB)
        return hA, cA, hB, cB

    carry0 = (h_ref[pl.ds(0, H), :], c_ref[pl.ds(0, H), :],
              h_ref[pl.ds(H, H), :], c_ref[pl.ds(H, H), :])
    hA, cA, hB, cB = jax.lax.fori_loop(0, CHUNK, step, carry0, unroll=2)
    h_ref[pl.ds(0, H), :] = hA
    c_ref[pl.ds(0, H), :] = cA
    h_ref[pl.ds(H, H), :] = hB
    c_ref[pl.ds(H, H), :] = cB

    @pl.when(t == n_t - 1)
    def _finish():
        wcls = wcls_ref[...]
        bcls = bcls_ref[...]
        out_ref[pl.ds(0, H), :] = (
            jnp.dot(hA, wcls, preferred_element_type=jnp.float32) + bcls
        ).astype(out_ref.dtype)
        out_ref[pl.ds(H, H), :] = (
            jnp.dot(hB, wcls, preferred_element_type=jnp.float32) + bcls
        ).astype(out_ref.dtype)


def kernel(x_ids, embedding, w_ih_T, w_hh_T, b_lstm, w_cls_T, b_cls):
    V, Dp = embedding.shape
    G = w_ih_T.shape[1]
    Cp = w_cls_T.shape[1]
    B, T = x_ids.shape

    if B % 256 == 0:
        TB = 256
    elif B % 16 == 0:
        TB = B
    else:
        TB = _round_up(B, 16)
    Bp = _round_up(B, TB)
    nb = Bp // TB
    CHUNK = _pick_chunk(T)
    nt = T // CHUNK

    emb_f32 = embedding.astype(jnp.float32)

    ids = x_ids
    if Bp != B:
        ids = jnp.pad(ids, ((0, Bp - B), (0, 0)))
    ids = ids.T.reshape(-1)

    full = lambda shape: pl.BlockSpec(shape, lambda b, t: tuple(0 for _ in shape))

    import functools
    out = pl.pallas_call(
        functools.partial(_lstm_cls_kernel, T),
        out_shape=jax.ShapeDtypeStruct((Bp, Cp), jnp.float32),
        grid=(nb, nt),
        in_specs=[
            pl.BlockSpec(memory_space=pltpu.SMEM),
            full((V, Dp)),
            full((Dp, G)),
            full((Dp, G)),
            full((1, G)),
            full((Dp, Cp)),
            full((1, Cp)),
        ],
        out_specs=pl.BlockSpec((TB, Cp), lambda b, t: (b, 0)),
        scratch_shapes=[
            pltpu.VMEM((TB, Dp), jnp.bfloat16),
            pltpu.VMEM((TB, Dp), jnp.float32),
            pltpu.VMEM((CHUNK * TB, Dp // 2), jnp.float32),
            pltpu.VMEM((CHUNK * TB, Dp // 2), jnp.float32),
        ],
        compiler_params=pltpu.CompilerParams(
            dimension_semantics=("parallel", "arbitrary"),
            vmem_limit_bytes=64 * 1024 * 1024,
        ),
    )(ids, emb_f32, w_ih_T, w_hh_T, b_lstm, w_cls_T, b_cls)
    return out[:B, :128]


# speedup vs baseline: 1.3157x; 1.0156x over previous
import jax
import jax.numpy as jnp
from jax import lax
from jax.experimental import pallas as pl
from jax.experimental.pallas import tpu as pltpu


def _round_up(x, m):
    return ((x + m - 1) // m) * m


def _pick_chunk(T, max_chunk=32):
    if T <= max_chunk:
        return T
    for c in range(max_chunk, 0, -1):
        if T % c == 0:
            return c
    return T


def _lstm_cls_kernel(t_len, ids_ref, emb_ref, wih_ref, whh_ref, b_ref,
                     wcls_ref, bcls_ref, out_ref, h_ref, c_ref,
                     xlo_ref, xhi_ref):
    b = pl.program_id(0)
    t = pl.program_id(1)
    n_t = pl.num_programs(1)
    TB, Dp = h_ref.shape
    CHUNK = xlo_ref.shape[0] // TB
    H = TB // 2
    DL = Dp // 2

    @pl.when(t == 0)
    def _init():
        h_ref[...] = jnp.zeros_like(h_ref)
        c_ref[...] = jnp.zeros_like(c_ref)

    n_b = pl.num_programs(0)
    fb0 = (t * CHUNK) * (n_b * TB) + b * TB

    def g_body(s, _):
        fb = fb0 + s * (n_b * TB)
        sTB = pl.multiple_of(s * TB, 8)
        for u in range(8):
          for g in range(TB // 8):
            j = g * 8 + u
            idx = ids_ref[fb + j]
            row0 = pl.multiple_of((idx >> 3) << 3, 8)
            ch = emb_ref[pl.ds(row0, 8), :]
            sh = u - (idx & 7)
            lo = pltpu.roll(ch[:, :DL], sh, axis=0)
            hi = pltpu.roll(ch[:, DL:], sh, axis=0)
            xlo_ref[pl.ds(sTB + j, 1), :] = lo[u:u + 1, :]
            xhi_ref[pl.ds(sTB + j, 1), :] = hi[u:u + 1, :]
        return 0

    lax.fori_loop(0, CHUNK, g_body, 0)

    wih = wih_ref[...]
    whh = whh_ref[...]
    bias = b_ref[...]

    def half_step(x_t, h, c):
        gates = (jnp.dot(x_t, wih, preferred_element_type=jnp.float32)
                 + jnp.dot(h, whh, preferred_element_type=jnp.float32)
                 + bias)
        sig = 0.5 * (1.0 + jnp.tanh(0.5 * gates[:, :3 * Dp]))
        g_g = jnp.tanh(gates[:, 3 * Dp:])
        i_g = sig[:, :Dp]
        f_g = sig[:, Dp:2 * Dp]
        o_g = sig[:, 2 * Dp:]
        c_new = f_g * c + i_g * g_g
        h_new = (o_g * jnp.tanh(c_new)).astype(jnp.bfloat16)
        return h_new, c_new

    def step(s, carry):
        hA, cA, hB, cB = carry
        base = pl.multiple_of(s * TB, 8)
        xA = jnp.concatenate([xlo_ref[pl.ds(base, H), :],
                              xhi_ref[pl.ds(base, H), :]], axis=1)
        xB = jnp.concatenate([xlo_ref[pl.ds(base + H, H), :],
                              xhi_ref[pl.ds(base + H, H), :]], axis=1)
        hA, cA = half_step(xA, hA, cA)
        hB, cB = half_step(xB, hB, cB)
        return hA, cA, hB, cB

    carry0 = (h_ref[pl.ds(0, H), :], c_ref[pl.ds(0, H), :],
              h_ref[pl.ds(H, H), :], c_ref[pl.ds(H, H), :])
    hA, cA, hB, cB = jax.lax.fori_loop(0, CHUNK, step, carry0, unroll=2)
    h_ref[pl.ds(0, H), :] = hA
    c_ref[pl.ds(0, H), :] = cA
    h_ref[pl.ds(H, H), :] = hB
    c_ref[pl.ds(H, H), :] = cB

    @pl.when(t == n_t - 1)
    def _finish():
        wcls = wcls_ref[...]
        bcls = bcls_ref[...]
        out_ref[pl.ds(0, H), :] = (
            jnp.dot(hA, wcls, preferred_element_type=jnp.float32) + bcls
        ).astype(out_ref.dtype)
        out_ref[pl.ds(H, H), :] = (
            jnp.dot(hB, wcls, preferred_element_type=jnp.float32) + bcls
        ).astype(out_ref.dtype)


def kernel(x_ids, embedding, w_ih_T, w_hh_T, b_lstm, w_cls_T, b_cls):
    V, Dp = embedding.shape
    G = w_ih_T.shape[1]
    Cp = w_cls_T.shape[1]
    B, T = x_ids.shape

    if B % 256 == 0:
        TB = 256
    elif B % 16 == 0:
        TB = B
    else:
        TB = _round_up(B, 16)
    Bp = _round_up(B, TB)
    nb = Bp // TB
    CHUNK = _pick_chunk(T)
    nt = T // CHUNK

    emb_f32 = embedding.astype(jnp.float32)

    ids = x_ids
    if Bp != B:
        ids = jnp.pad(ids, ((0, Bp - B), (0, 0)))
    ids = ids.T.reshape(-1)

    full = lambda shape: pl.BlockSpec(shape, lambda b, t: tuple(0 for _ in shape))

    import functools
    out = pl.pallas_call(
        functools.partial(_lstm_cls_kernel, T),
        out_shape=jax.ShapeDtypeStruct((Bp, Cp), jnp.float32),
        grid=(nb, nt),
        in_specs=[
            pl.BlockSpec(memory_space=pltpu.SMEM),
            full((V, Dp)),
            full((Dp, G)),
            full((Dp, G)),
            full((1, G)),
            full((Dp, Cp)),
            full((1, Cp)),
        ],
        out_specs=pl.BlockSpec((TB, Cp), lambda b, t: (b, 0)),
        scratch_shapes=[
            pltpu.VMEM((TB, Dp), jnp.bfloat16),
            pltpu.VMEM((TB, Dp), jnp.float32),
            pltpu.VMEM((CHUNK * TB, Dp // 2), jnp.float32),
            pltpu.VMEM((CHUNK * TB, Dp // 2), jnp.float32),
        ],
        compiler_params=pltpu.CompilerParams(
            dimension_semantics=("parallel", "arbitrary"),
            vmem_limit_bytes=64 * 1024 * 1024,
        ),
    )(ids, emb_f32, w_ih_T, w_hh_T, b_lstm, w_cls_T, b_cls)
    return out[:B, :128]
